# baseline (speedup 1.0000x reference)
import functools

import jax
import jax.numpy as jnp
from jax import lax
from jax.experimental import pallas as pl
from jax.experimental.pallas import tpu as pltpu
from jax.experimental.pallas import tpu_sc as plsc


def _prep_body(ids_ref, state_ref, species_ref, fid_ref, comb_ref):
    comb_ref[...] = species_ref[...][:, None, :] + state_ref[...][None, :, :]
    ids = ids_ref[...]
    bs = lax.broadcasted_iota(jnp.int32, ids.shape, 0)
    s_idx = lax.rem(bs, jnp.int32(species_ref.shape[0]))
    fid_ref[...] = ids + 3 * s_idx


def _make_sc_gather(n_rows, h, per_w, chunk, num_cores):
    nch = per_w // chunk
    mesh = plsc.VectorSubcoreMesh(core_axis_name="c", subcore_axis_name="s")

    @functools.partial(
        pl.kernel,
        mesh=mesh,
        out_type=jax.ShapeDtypeStruct((n_rows, h), jnp.float32),
        scratch_types=[
            pltpu.VMEM((per_w,), jnp.int32),
            pltpu.VMEM((chunk, h), jnp.float32),
            pltpu.VMEM((chunk, h), jnp.float32),
            pltpu.SemaphoreType.DMA,
            pltpu.SemaphoreType.DMA,
            pltpu.SemaphoreType.DMA,
            pltpu.SemaphoreType.DMA,
        ],
    )
    def sc_gather(fid_hbm, comb_hbm, out_hbm, idx_v, buf0, buf1, sg0, sg1, ss0, ss1):
        wid = lax.axis_index("s") * num_cores + lax.axis_index("c")
        base = wid * per_w
        pltpu.sync_copy(fid_hbm.at[pl.ds(base, per_w)], idx_v)

        bufs = (buf0, buf1)
        gsems = (sg0, sg1)
        ssems = (ss0, ss1)

        def start_gather(c):
            b = c % 2
            return pltpu.async_copy(
                comb_hbm.at[idx_v.at[pl.ds(c * chunk, chunk)]], bufs[b], gsems[b]
            )

        def start_scatter(c):
            b = c % 2
            return pltpu.async_copy(
                bufs[b], out_hbm.at[pl.ds(base + c * chunk, chunk)], ssems[b]
            )

        g = [None] * nch
        s = [None] * nch
        for c in range(nch):
            if c >= 2:
                s[c - 2].wait()
            g[c] = start_gather(c)
            if c >= 1:
                g[c - 1].wait()
                s[c - 1] = start_scatter(c - 1)
        g[nch - 1].wait()
        s[nch - 1] = start_scatter(nch - 1)
        if nch >= 2:
            s[nch - 2].wait()
        s[nch - 1].wait()

    return sc_gather


def kernel(input_ids, state_table, species_table):
    B, S, T = input_ids.shape
    K, H = state_table.shape
    ids2d = input_ids.reshape(B * S, T).astype(jnp.int32)

    fid2d, comb3d = pl.pallas_call(
        _prep_body,
        out_shape=(
            jax.ShapeDtypeStruct((B * S, T), jnp.int32),
            jax.ShapeDtypeStruct((S, K, H), jnp.float32),
        ),
    )(ids2d, state_table, species_table)

    fid = fid2d.reshape(B * S * T)
    comb = comb3d.reshape(S * K, H)

    n_rows = B * S * T
    info = plsc.get_sparse_core_info()
    nw = info.num_cores * info.num_subcores
    per_w = n_rows // nw
    chunk = 200

    sc_gather = _make_sc_gather(n_rows, H, per_w, chunk, info.num_cores)
    out = sc_gather(fid, comb)
    return out.reshape(B, S, T, H)


# speedup vs baseline: 1.3522x; 1.3522x over previous
import functools

import jax
import jax.numpy as jnp
from jax import lax
from jax.experimental import pallas as pl
from jax.experimental.pallas import tpu as pltpu
from jax.experimental.pallas import tpu_sc as plsc


def _prep_body(ids_ref, state_ref, species_ref, fid_ref, comb_ref):
    comb_ref[...] = species_ref[...][:, None, :] + state_ref[...][None, :, :]
    ids = ids_ref[...]
    bs = lax.broadcasted_iota(jnp.int32, ids.shape, 0)
    s_idx = lax.rem(bs, jnp.int32(species_ref.shape[0]))
    fid = ids + 3 * s_idx
    pad = fid_ref.shape[1] - ids.shape[1]
    fid_ref[...] = jnp.concatenate(
        [fid, jnp.zeros((ids.shape[0], pad), jnp.int32)], axis=1
    )


def _make_sc_gather(n_bs, t, tp, h, bs_per_w, bs_chunk, num_cores):
    nch = bs_per_w // bs_chunk
    per_w = bs_per_w * tp
    mesh = plsc.VectorSubcoreMesh(core_axis_name="c", subcore_axis_name="s")

    @functools.partial(
        pl.kernel,
        mesh=mesh,
        out_type=jax.ShapeDtypeStruct((n_bs, t, h), jnp.float32),
        scratch_types=[
            pltpu.VMEM((per_w,), jnp.int32),
            pltpu.VMEM((bs_chunk, t, h), jnp.float32),
            pltpu.VMEM((bs_chunk, t, h), jnp.float32),
            pltpu.SemaphoreType.DMA,
            pltpu.SemaphoreType.DMA,
            pltpu.SemaphoreType.DMA,
            pltpu.SemaphoreType.DMA,
        ],
    )
    def sc_gather(fid_hbm, comb_hbm, out_hbm, idx_v, buf0, buf1, sg0, sg1, ss0, ss1):
        wid = lax.axis_index("s") * num_cores + lax.axis_index("c")
        base = wid * per_w
        bs_base = wid * bs_per_w
        pltpu.sync_copy(fid_hbm.at[pl.ds(base, per_w)], idx_v)

        bufs = (buf0, buf1)
        gsems = (sg0, sg1)
        ssems = (ss0, ss1)

        def start_gather(c):
            b = c % 2
            hs = []
            for j in range(bs_chunk):
                hs.append(
                    pltpu.async_copy(
                        comb_hbm.at[idx_v.at[pl.ds((c * bs_chunk + j) * tp, t)]],
                        bufs[b].at[j],
                        gsems[b],
                    )
                )
            return hs

        def start_scatter(c):
            b = c % 2
            return pltpu.async_copy(
                bufs[b], out_hbm.at[pl.ds(bs_base + c * bs_chunk, bs_chunk)], ssems[b]
            )

        def wait_all(hs):
            for hh in hs:
                hh.wait()

        g = [None] * nch
        s = [None] * nch
        for c in range(nch):
            if c >= 2:
                s[c - 2].wait()
            g[c] = start_gather(c)
            if c >= 1:
                wait_all(g[c - 1])
                s[c - 1] = start_scatter(c - 1)
        wait_all(g[nch - 1])
        s[nch - 1] = start_scatter(nch - 1)
        if nch >= 2:
            s[nch - 2].wait()
        s[nch - 1].wait()

    return sc_gather


def kernel(input_ids, state_table, species_table):
    B, S, T = input_ids.shape
    K, H = state_table.shape
    ids2d = input_ids.reshape(B * S, T).astype(jnp.int32)

    TP = (T + 7) // 8 * 8
    fid2d, comb3d = pl.pallas_call(
        _prep_body,
        out_shape=(
            jax.ShapeDtypeStruct((B * S, TP), jnp.int32),
            jax.ShapeDtypeStruct((S, K, H), jnp.float32),
        ),
    )(ids2d, state_table, species_table)

    fid = fid2d.reshape(B * S * TP)
    comb = comb3d.reshape(S * K, H)

    n_bs = B * S
    info = plsc.get_sparse_core_info()
    nw = info.num_cores * info.num_subcores
    bs_per_w = n_bs // nw
    bs_chunk = 4

    sc_gather = _make_sc_gather(n_bs, T, TP, H, bs_per_w, bs_chunk, info.num_cores)
    out = sc_gather(fid, comb)
    return out.reshape(B, S, T, H)


# speedup vs baseline: 2.7178x; 2.0099x over previous
import jax
import jax.numpy as jnp
from jax.experimental import pallas as pl


def _tc_body(ids_ref, state_ref, species_ref, out_ref):
    ids = ids_ref[...]
    st = state_ref[...]
    sp = species_ref[...]
    idsx = ids[:, :, None]
    r0 = st[0][None, None, :]
    r1 = st[1][None, None, :]
    r2 = st[2][None, None, :]
    state_emb = jnp.where(idsx == 0, r0, jnp.where(idsx == 1, r1, r2))
    out_ref[...] = state_emb + sp[:, None, :]


def kernel(input_ids, state_table, species_table):
    B, S, T = input_ids.shape
    H = state_table.shape[1]
    ids = input_ids.reshape(B * S, T).astype(jnp.int32)
    Sb = 32
    grid = (B * S) // Sb
    s_blocks = S // Sb
    out = pl.pallas_call(
        _tc_body,
        grid=(grid,),
        in_specs=[
            pl.BlockSpec((Sb, T), lambda i: (i, 0)),
            pl.BlockSpec((3, H), lambda i: (0, 0)),
            pl.BlockSpec((Sb, H), lambda i: (i % s_blocks, 0)),
        ],
        out_specs=pl.BlockSpec((Sb, T, H), lambda i: (i, 0, 0)),
        out_shape=jax.ShapeDtypeStruct((B * S, T, H), jnp.float32),
    )(ids, state_table, species_table)
    return out.reshape(B, S, T, H)


# speedup vs baseline: 3.1254x; 1.1500x over previous
import jax
import jax.numpy as jnp
from jax.experimental import pallas as pl


def _tc_body(ids_ref, state_ref, species_ref, out_ref):
    ids = ids_ref[...]
    st = state_ref[...]
    sp = species_ref[...]
    idsx = ids[0][:, :, None]
    r0 = st[0][None, None, :]
    r1 = st[1][None, None, :]
    r2 = st[2][None, None, :]
    state_emb = jnp.where(idsx == 0, r0, jnp.where(idsx == 1, r1, r2))
    out_ref[...] = (state_emb + sp[:, None, :])[None]


def kernel(input_ids, state_table, species_table):
    B, S, T = input_ids.shape
    H = state_table.shape[1]
    ids = input_ids.astype(jnp.int32)
    Sb = 32
    s_blocks = S // Sb
    out = pl.pallas_call(
        _tc_body,
        grid=(B, s_blocks),
        in_specs=[
            pl.BlockSpec((1, Sb, T), lambda b, j: (b, j, 0)),
            pl.BlockSpec((3, H), lambda b, j: (0, 0)),
            pl.BlockSpec((Sb, H), lambda b, j: (j, 0)),
        ],
        out_specs=pl.BlockSpec((1, Sb, T, H), lambda b, j: (b, j, 0, 0)),
        out_shape=jax.ShapeDtypeStruct((B, S, T, H), jnp.float32),
    )(ids, state_table, species_table)
    return out


# speedup vs baseline: 6.6885x; 2.1401x over previous
import jax
import jax.numpy as jnp
from jax.experimental import pallas as pl


def _tc_body(ids_ref, state_ref, species_ref, out_ref):
    ids = ids_ref[...]
    st = state_ref[...]
    sp = species_ref[...]
    ids_t = jnp.transpose(ids[0], (1, 0))
    idsx = ids_t[:, :, None]
    r0 = st[0][None, None, :]
    r1 = st[1][None, None, :]
    r2 = st[2][None, None, :]
    state_emb = jnp.where(idsx == 0, r0, jnp.where(idsx == 1, r1, r2))
    out_ref[...] = (state_emb + sp[None, :, :])[None]


def kernel(input_ids, state_table, species_table):
    B, S, T = input_ids.shape
    H = state_table.shape[1]
    ids = input_ids.astype(jnp.int32)
    Sb = 32
    s_blocks = S // Sb
    out_t = pl.pallas_call(
        _tc_body,
        grid=(B, s_blocks),
        in_specs=[
            pl.BlockSpec((1, Sb, T), lambda b, j: (b, j, 0)),
            pl.BlockSpec((3, H), lambda b, j: (0, 0)),
            pl.BlockSpec((Sb, H), lambda b, j: (j, 0)),
        ],
        out_specs=pl.BlockSpec((1, T, Sb, H), lambda b, j: (b, 0, j, 0)),
        out_shape=jax.ShapeDtypeStruct((B, T, S, H), jnp.float32),
    )(ids, state_table, species_table)
    return jnp.swapaxes(out_t, 1, 2)


# speedup vs baseline: 9.2858x; 1.3883x over previous
import jax
import jax.numpy as jnp
from jax.experimental import pallas as pl


def _tc_body(ids_ref, state_ref, species_ref, out_ref):
    ids = ids_ref[...]
    st = state_ref[...]
    sp = species_ref[...]
    ids_t = jnp.transpose(ids[0], (1, 0))
    idsx = ids_t[:, :, None]
    r0 = st[0][None, None, :]
    r1 = st[1][None, None, :]
    r2 = st[2][None, None, :]
    state_emb = jnp.where(idsx == 0, r0, jnp.where(idsx == 1, r1, r2))
    out_ref[...] = (state_emb + sp[None, :, :])[None]


def kernel(input_ids, state_table, species_table):
    B, S, T = input_ids.shape
    H = state_table.shape[1]
    ids = input_ids.astype(jnp.int32)
    Sb = 64
    s_blocks = S // Sb
    out_t = pl.pallas_call(
        _tc_body,
        grid=(B, s_blocks),
        in_specs=[
            pl.BlockSpec((1, Sb, T), lambda b, j: (b, j, 0)),
            pl.BlockSpec((3, H), lambda b, j: (0, 0)),
            pl.BlockSpec((Sb, H), lambda b, j: (j, 0)),
        ],
        out_specs=pl.BlockSpec((1, T, Sb, H), lambda b, j: (b, 0, j, 0)),
        out_shape=jax.ShapeDtypeStruct((B, T, S, H), jnp.float32),
    )(ids, state_table, species_table)
    return jnp.swapaxes(out_t, 1, 2)


# speedup vs baseline: 10.8688x; 1.1705x over previous
import jax
import jax.numpy as jnp
from jax.experimental import pallas as pl


def _tc_body(ids_ref, state_ref, species_ref, out_ref):
    ids = ids_ref[...]
    st = state_ref[...]
    sp = species_ref[...]
    ids_t = jnp.transpose(ids[0], (1, 0))
    idsx = ids_t[:, :, None]
    r0 = st[0][None, None, :]
    r1 = st[1][None, None, :]
    r2 = st[2][None, None, :]
    state_emb = jnp.where(idsx == 0, r0, jnp.where(idsx == 1, r1, r2))
    out_ref[...] = (state_emb + sp[None, :, :])[None]


def kernel(input_ids, state_table, species_table):
    B, S, T = input_ids.shape
    H = state_table.shape[1]
    ids = input_ids.astype(jnp.int32)
    Sb = 128
    s_blocks = S // Sb
    out_t = pl.pallas_call(
        _tc_body,
        grid=(B, s_blocks),
        in_specs=[
            pl.BlockSpec((1, Sb, T), lambda b, j: (b, j, 0)),
            pl.BlockSpec((3, H), lambda b, j: (0, 0)),
            pl.BlockSpec((Sb, H), lambda b, j: (j, 0)),
        ],
        out_specs=pl.BlockSpec((1, T, Sb, H), lambda b, j: (b, 0, j, 0)),
        out_shape=jax.ShapeDtypeStruct((B, T, S, H), jnp.float32),
    )(ids, state_table, species_table)
    return jnp.swapaxes(out_t, 1, 2)


# speedup vs baseline: 10.9848x; 1.0107x over previous
import jax
import jax.numpy as jnp
from jax.experimental import pallas as pl


def _tc_body(ids_ref, state_ref, species_ref, out_ref):
    ids = ids_ref[...]
    st = state_ref[...]
    sp = species_ref[...]
    ids_t = jnp.transpose(ids[0], (1, 0))
    idsx = ids_t[:, :, None]
    r0 = st[0][None, None, :]
    r1 = st[1][None, None, :]
    r2 = st[2][None, None, :]
    state_emb = jnp.where(idsx == 0, r0, jnp.where(idsx == 1, r1, r2))
    out_ref[...] = (state_emb + sp[None, :, :])[None]


def kernel(input_ids, state_table, species_table):
    B, S, T = input_ids.shape
    H = state_table.shape[1]
    ids = input_ids.astype(jnp.int32)
    Sb = 256
    s_blocks = S // Sb
    out_t = pl.pallas_call(
        _tc_body,
        grid=(B, s_blocks),
        in_specs=[
            pl.BlockSpec((1, Sb, T), lambda b, j: (b, j, 0)),
            pl.BlockSpec((3, H), lambda b, j: (0, 0)),
            pl.BlockSpec((Sb, H), lambda b, j: (j, 0)),
        ],
        out_specs=pl.BlockSpec((1, T, Sb, H), lambda b, j: (b, 0, j, 0)),
        out_shape=jax.ShapeDtypeStruct((B, T, S, H), jnp.float32),
    )(ids, state_table, species_table)
    return jnp.swapaxes(out_t, 1, 2)
